# baseline (speedup 1.0000x reference)
import functools

import jax
import jax.numpy as jnp
from jax import lax
from jax.experimental import pallas as pl
from jax.experimental.pallas import tpu as pltpu
from jax.experimental.pallas import tpu_sc as plsc

_NUM_CORES = 2
_NUM_SUBCORES = 16
_NUM_WORKERS = _NUM_CORES * _NUM_SUBCORES


@functools.partial(jax.jit, static_argnums=(1,))
def _sc_copy(flat, total):
    chunk = ((total + _NUM_WORKERS - 1) // _NUM_WORKERS + 7) // 8 * 8
    nfull = total // chunk
    tail = total - nfull * chunk

    mesh = plsc.VectorSubcoreMesh(core_axis_name="c", subcore_axis_name="s")

    half = chunk // 2

    @functools.partial(
        pl.kernel,
        mesh=mesh,
        out_type=jax.ShapeDtypeStruct((total,), jnp.float32),
        scratch_types=[
            pltpu.VMEM((chunk,), jnp.float32),
            pltpu.SemaphoreType.DMA,
            pltpu.SemaphoreType.DMA,
        ],
    )
    def _copy(flat_hbm, out_hbm, buf, sem_a, sem_b):
        wid = lax.axis_index("s") * _NUM_CORES + lax.axis_index("c")
        base = wid * chunk

        @pl.when(wid < nfull)
        def _():
            in_a = pltpu.async_copy(
                flat_hbm.at[pl.ds(base, half)], buf.at[pl.ds(0, half)], sem_a
            )
            in_b = pltpu.async_copy(
                flat_hbm.at[pl.ds(base + half, chunk - half)],
                buf.at[pl.ds(half, chunk - half)],
                sem_b,
            )
            in_a.wait()
            out_a = pltpu.async_copy(
                buf.at[pl.ds(0, half)], out_hbm.at[pl.ds(base, half)], sem_a
            )
            in_b.wait()
            out_b = pltpu.async_copy(
                buf.at[pl.ds(half, chunk - half)],
                out_hbm.at[pl.ds(base + half, chunk - half)],
                sem_b,
            )
            out_a.wait()
            out_b.wait()

        if tail:

            @pl.when(wid == nfull)
            def _():
                tbase = nfull * chunk
                pltpu.sync_copy(
                    flat_hbm.at[pl.ds(tbase, tail)], buf.at[pl.ds(0, tail)]
                )
                pltpu.sync_copy(
                    buf.at[pl.ds(0, tail)], out_hbm.at[pl.ds(tbase, tail)]
                )

    return _copy(flat)


def kernel(flat, row_lengths):
    out_flat = _sc_copy(flat.astype(jnp.float32), flat.shape[0])
    return out_flat, row_lengths


# speedup vs baseline: 1.0061x; 1.0061x over previous
import functools

import jax
import jax.numpy as jnp
from jax import lax
from jax.experimental import pallas as pl
from jax.experimental.pallas import tpu as pltpu
from jax.experimental.pallas import tpu_sc as plsc

_NUM_CORES = 2
_NUM_SUBCORES = 16
_NUM_WORKERS = _NUM_CORES * _NUM_SUBCORES


@functools.partial(jax.jit, static_argnums=(1,))
def _sc_copy(flat, total):
    chunk = ((total + _NUM_WORKERS - 1) // _NUM_WORKERS + 7) // 8 * 8
    nfull = total // chunk
    tail = total - nfull * chunk

    mesh = plsc.VectorSubcoreMesh(core_axis_name="c", subcore_axis_name="s")

    @functools.partial(
        pl.kernel,
        mesh=mesh,
        out_type=jax.ShapeDtypeStruct((total,), jnp.float32),
        scratch_types=[pltpu.VMEM((chunk,), jnp.float32)],
    )
    def _copy(flat_hbm, out_hbm, buf):
        wid = lax.axis_index("s") * _NUM_CORES + lax.axis_index("c")
        base = wid * chunk

        @pl.when(wid < nfull)
        def _():
            pltpu.sync_copy(flat_hbm.at[pl.ds(base, chunk)], buf)
            pltpu.sync_copy(buf, out_hbm.at[pl.ds(base, chunk)])

        if tail:

            @pl.when(wid == nfull)
            def _():
                tbase = nfull * chunk
                pltpu.sync_copy(
                    flat_hbm.at[pl.ds(tbase, tail)], buf.at[pl.ds(0, tail)]
                )
                pltpu.sync_copy(
                    buf.at[pl.ds(0, tail)], out_hbm.at[pl.ds(tbase, tail)]
                )

    return _copy(flat)


def kernel(flat, row_lengths):
    out_flat = _sc_copy(flat.astype(jnp.float32), flat.shape[0])
    return out_flat, row_lengths
